# baseline (speedup 1.0000x reference)
import functools

import jax
import jax.numpy as jnp
from jax import lax
from jax.experimental import pallas as pl
from jax.experimental.pallas import tpu as pltpu
from jax.experimental.pallas import tpu_sc as plsc

B, T, C, H, W = 4, 32, 3, 224, 224
S = max(1, T // 4)
ROW = C * H * W
NCHUNK = 2
CH = ROW // NCHUNK

_NC = 2
_NS = 16
_NW = _NC * _NS

_mesh = plsc.VectorSubcoreMesh(core_axis_name="c", subcore_axis_name="s")


@functools.partial(
    pl.kernel,
    out_type=jax.ShapeDtypeStruct((B, S, C, H, W), jnp.float32),
    mesh=_mesh,
    scratch_types=[
        pltpu.VMEM((H, W), jnp.float32),
        pltpu.VMEM((H, W), jnp.float32),
        pltpu.SemaphoreType.DMA,
        pltpu.SemaphoreType.DMA,
        pltpu.SemaphoreType.DMA,
        pltpu.SemaphoreType.DMA,
    ],
)
def _slow_gather(frames_hbm, out_hbm, buf0, buf1, si0, si1, so0, so1):
    wid = lax.axis_index("s") * _NC + lax.axis_index("c")
    b = wid // S
    s = wid % S
    src_t = (s * (T - 1)) // (S - 1)
    in0 = pltpu.async_copy(frames_hbm.at[b, src_t, 0], buf0, si0)
    in1 = pltpu.async_copy(frames_hbm.at[b, src_t, 1], buf1, si1)
    in0.wait()
    out0 = pltpu.async_copy(buf0, out_hbm.at[b, s, 0], so0)
    in1.wait()
    out1 = pltpu.async_copy(buf1, out_hbm.at[b, s, 1], so1)
    out0.wait()
    in2 = pltpu.async_copy(frames_hbm.at[b, src_t, 2], buf0, si0)
    in2.wait()
    out2 = pltpu.async_copy(buf0, out_hbm.at[b, s, 2], so0)
    out1.wait()
    out2.wait()


_NDMA = 8
_TW = T // _NDMA


def _fast_copy_body(x_hbm, o_hbm, sems):
    cps = [
        pltpu.make_async_copy(
            x_hbm.at[:, pl.ds(i * _TW, _TW)],
            o_hbm.at[:, pl.ds(i * _TW, _TW)],
            sems.at[i],
        )
        for i in range(_NDMA)
    ]
    for cp in cps:
        cp.start()
    for cp in cps:
        cp.wait()


def _fast_copy(frames):
    return pl.pallas_call(
        _fast_copy_body,
        in_specs=[pl.BlockSpec(memory_space=pltpu.MemorySpace.HBM)],
        out_specs=pl.BlockSpec(memory_space=pltpu.MemorySpace.HBM),
        out_shape=jax.ShapeDtypeStruct((B, T, C, H, W), jnp.float32),
        scratch_shapes=[pltpu.SemaphoreType.DMA((_NDMA,))],
    )(frames)


def kernel(frames):
    return (_slow_gather(frames), _fast_copy(frames))


# speedup vs baseline: 1.0007x; 1.0007x over previous
import functools

import jax
import jax.numpy as jnp
from jax import lax
from jax.experimental import pallas as pl
from jax.experimental.pallas import tpu as pltpu
from jax.experimental.pallas import tpu_sc as plsc

B, T, C, H, W = 4, 32, 3, 224, 224
S = max(1, T // 4)
ROW = C * H * W
NCHUNK = 2
CH = ROW // NCHUNK

_NC = 2
_NS = 16
_NW = _NC * _NS

_mesh = plsc.VectorSubcoreMesh(core_axis_name="c", subcore_axis_name="s")


@functools.partial(
    pl.kernel,
    out_type=jax.ShapeDtypeStruct((B, S, C, H, W), jnp.float32),
    mesh=_mesh,
    scratch_types=[
        pltpu.VMEM((H, W), jnp.float32),
        pltpu.VMEM((H, W), jnp.float32),
        pltpu.SemaphoreType.DMA,
        pltpu.SemaphoreType.DMA,
        pltpu.SemaphoreType.DMA,
        pltpu.SemaphoreType.DMA,
    ],
)
def _slow_gather(frames_hbm, out_hbm, buf0, buf1, si0, si1, so0, so1):
    wid = lax.axis_index("s") * _NC + lax.axis_index("c")
    b = wid // S
    s = wid % S
    src_t = (s * (T - 1)) // (S - 1)
    in0 = pltpu.async_copy(frames_hbm.at[b, src_t, 0], buf0, si0)
    in1 = pltpu.async_copy(frames_hbm.at[b, src_t, 1], buf1, si1)
    in0.wait()
    out0 = pltpu.async_copy(buf0, out_hbm.at[b, s, 0], so0)
    in1.wait()
    out1 = pltpu.async_copy(buf1, out_hbm.at[b, s, 1], so1)
    out0.wait()
    in2 = pltpu.async_copy(frames_hbm.at[b, src_t, 2], buf0, si0)
    in2.wait()
    out2 = pltpu.async_copy(buf0, out_hbm.at[b, s, 2], so0)
    out1.wait()
    out2.wait()


_NDMA = 8
_TW = T // 2


def _fast_copy_body(x_hbm, o_hbm, sems):
    cps = [
        pltpu.make_async_copy(
            x_hbm.at[i // 2, pl.ds((i % 2) * _TW, _TW)],
            o_hbm.at[i // 2, pl.ds((i % 2) * _TW, _TW)],
            sems.at[i],
        )
        for i in range(_NDMA)
    ]
    for cp in cps:
        cp.start()
    for cp in cps:
        cp.wait()


def _fast_copy(frames):
    return pl.pallas_call(
        _fast_copy_body,
        in_specs=[pl.BlockSpec(memory_space=pltpu.MemorySpace.HBM)],
        out_specs=pl.BlockSpec(memory_space=pltpu.MemorySpace.HBM),
        out_shape=jax.ShapeDtypeStruct((B, T, C, H, W), jnp.float32),
        scratch_shapes=[pltpu.SemaphoreType.DMA((_NDMA,))],
    )(frames)


def kernel(frames):
    return (_slow_gather(frames), _fast_copy(frames))


# speedup vs baseline: 30.1007x; 30.0808x over previous
import functools

import jax
import jax.numpy as jnp
from jax import lax
from jax.experimental import pallas as pl
from jax.experimental.pallas import tpu as pltpu
from jax.experimental.pallas import tpu_sc as plsc

B, T, C, H, W = 4, 32, 3, 224, 224
S = max(1, T // 4)
ROW = C * H * W
NCHUNK = 2
CH = ROW // NCHUNK

_NC = 2
_NS = 16
_NW = _NC * _NS

_mesh = plsc.VectorSubcoreMesh(core_axis_name="c", subcore_axis_name="s")


@functools.partial(
    pl.kernel,
    out_type=jax.ShapeDtypeStruct((B, S, C, H, W), jnp.float32),
    mesh=_mesh,
    scratch_types=[
        pltpu.VMEM((H, W), jnp.float32),
        pltpu.VMEM((H, W), jnp.float32),
        pltpu.SemaphoreType.DMA,
        pltpu.SemaphoreType.DMA,
        pltpu.SemaphoreType.DMA,
        pltpu.SemaphoreType.DMA,
    ],
)
def _slow_gather(frames_hbm, out_hbm, buf0, buf1, si0, si1, so0, so1):
    wid = lax.axis_index("s") * _NC + lax.axis_index("c")
    b = wid // S
    s = wid % S
    src_t = (s * (T - 1)) // (S - 1)
    in0 = pltpu.async_copy(frames_hbm.at[b, src_t, 0], buf0, si0)
    in1 = pltpu.async_copy(frames_hbm.at[b, src_t, 1], buf1, si1)
    in0.wait()
    out0 = pltpu.async_copy(buf0, out_hbm.at[b, s, 0], so0)
    in1.wait()
    out1 = pltpu.async_copy(buf1, out_hbm.at[b, s, 1], so1)
    out0.wait()
    in2 = pltpu.async_copy(frames_hbm.at[b, src_t, 2], buf0, si0)
    in2.wait()
    out2 = pltpu.async_copy(buf0, out_hbm.at[b, s, 2], so0)
    out1.wait()
    out2.wait()


_FBLK = 4
_NBUF = 6
_NBLK = (B * T) // _FBLK
_BPB = T // _FBLK


def _fast_copy_body(x_hbm, o_hbm, *bufs_and_sems):
    bufs = bufs_and_sems[:_NBUF]
    si, so = bufs_and_sems[_NBUF], bufs_and_sems[_NBUF + 1]

    def start_in(n):
        return pltpu.async_copy(
            x_hbm.at[n // _BPB, pl.ds((n % _BPB) * _FBLK, _FBLK)],
            bufs[n % _NBUF], si.at[n % _NBUF])

    def start_out(n):
        return pltpu.async_copy(
            bufs[n % _NBUF],
            o_hbm.at[n // _BPB, pl.ds((n % _BPB) * _FBLK, _FBLK)],
            so.at[n % _NBUF])

    ins, outs = {}, {}
    for n in range(min(_NBUF, _NBLK)):
        ins[n] = start_in(n)
    for n in range(_NBLK):
        if n >= 1:
            outs[n - 1].wait()
            if n + _NBUF - 1 < _NBLK:
                ins[n + _NBUF - 1] = start_in(n + _NBUF - 1)
        ins[n].wait()
        outs[n] = start_out(n)
    outs[_NBLK - 1].wait()


def _fast_copy(frames):
    return pl.pallas_call(
        _fast_copy_body,
        in_specs=[pl.BlockSpec(memory_space=pltpu.MemorySpace.HBM)],
        out_specs=pl.BlockSpec(memory_space=pltpu.MemorySpace.HBM),
        out_shape=jax.ShapeDtypeStruct((B, T, C, H, W), jnp.float32),
        scratch_shapes=(
            [pltpu.VMEM((_FBLK, C, H, W), jnp.float32)] * _NBUF
            + [pltpu.SemaphoreType.DMA((_NBUF,)),
               pltpu.SemaphoreType.DMA((_NBUF,))]
        ),
    )(frames)


def kernel(frames):
    return (_slow_gather(frames), _fast_copy(frames))


# speedup vs baseline: 31.5829x; 1.0492x over previous
import functools

import jax
import jax.numpy as jnp
from jax import lax
from jax.experimental import pallas as pl
from jax.experimental.pallas import tpu as pltpu
from jax.experimental.pallas import tpu_sc as plsc

B, T, C, H, W = 4, 32, 3, 224, 224
S = max(1, T // 4)
ROW = C * H * W
NCHUNK = 2
CH = ROW // NCHUNK

_NC = 2
_NS = 16
_NW = _NC * _NS

_mesh = plsc.VectorSubcoreMesh(core_axis_name="c", subcore_axis_name="s")


@functools.partial(
    pl.kernel,
    out_type=jax.ShapeDtypeStruct((B, S, C, H, W), jnp.float32),
    mesh=_mesh,
    scratch_types=[
        pltpu.VMEM((H, W), jnp.float32),
        pltpu.VMEM((H, W), jnp.float32),
        pltpu.SemaphoreType.DMA,
        pltpu.SemaphoreType.DMA,
        pltpu.SemaphoreType.DMA,
        pltpu.SemaphoreType.DMA,
    ],
)
def _slow_gather(frames_hbm, out_hbm, buf0, buf1, si0, si1, so0, so1):
    wid = lax.axis_index("s") * _NC + lax.axis_index("c")
    b = wid // S
    s = wid % S
    src_t = (s * (T - 1)) // (S - 1)
    in0 = pltpu.async_copy(frames_hbm.at[b, src_t, 0], buf0, si0)
    in1 = pltpu.async_copy(frames_hbm.at[b, src_t, 1], buf1, si1)
    in0.wait()
    out0 = pltpu.async_copy(buf0, out_hbm.at[b, s, 0], so0)
    in1.wait()
    out1 = pltpu.async_copy(buf1, out_hbm.at[b, s, 1], so1)
    out0.wait()
    in2 = pltpu.async_copy(frames_hbm.at[b, src_t, 2], buf0, si0)
    in2.wait()
    out2 = pltpu.async_copy(buf0, out_hbm.at[b, s, 2], so0)
    out1.wait()
    out2.wait()


def _fast_copy_body(x_ref, o_ref):
    o_ref[...] = x_ref[...]


_TBLK = 16


def _fast_copy(frames):
    return pl.pallas_call(
        _fast_copy_body,
        grid=(B, T // _TBLK),
        in_specs=[pl.BlockSpec((1, _TBLK, C, H, W), lambda i, j: (i, j, 0, 0, 0))],
        out_specs=pl.BlockSpec((1, _TBLK, C, H, W), lambda i, j: (i, j, 0, 0, 0)),
        out_shape=jax.ShapeDtypeStruct((B, T, C, H, W), jnp.float32),
    )(frames)


def kernel(frames):
    return (_slow_gather(frames), _fast_copy(frames))
